# baseline (speedup 1.0000x reference)
import functools

import jax
import jax.numpy as jnp
from jax import lax
from jax.experimental import pallas as pl
from jax.experimental.pallas import tpu as pltpu
from jax.experimental.pallas import tpu_sc as plsc

B = 16384
E = 64
F = 8

_NC = 2
_NS = 16
_NW = _NC * _NS
_BPW = B // _NW


_NB = 3
_LA = 2


def _emb_body(id_h, age_h, pvalue_h, shop_h, occu_h, city_h, gender_h, cms_h,
              w_id_h, w_age_h, w_pvalue_h, w_shop_h, w_occu_h, w_city_h,
              w_gender_h, w_cms_h, out_h, idx_v, bufs_v, isem, gsems, wsems):
    wid = lax.axis_index("s") * _NC + lax.axis_index("c")
    base = wid * _BPW
    idx_hbm = (id_h, age_h, pvalue_h, shop_h, occu_h, city_h, gender_h, cms_h)
    tables = (w_id_h, w_age_h, w_pvalue_h, w_shop_h, w_occu_h, w_city_h,
              w_gender_h, w_cms_h)

    icopies = [pltpu.async_copy(idx_hbm[f].at[pl.ds(base, _BPW)],
                                idx_v.at[f], isem) for f in range(F)]
    for c in icopies:
        c.wait()

    gd = [None] * F
    wd = [None] * F

    ABLATE_GATHER = False
    ABLATE_WRITE = True

    def start_writeback(t):
        b = t % _NB
        if ABLATE_WRITE:
            wd[t] = None
            return
        gd[t].wait()
        wd[t] = pltpu.async_copy(
            bufs_v.at[b],
            out_h.at[pl.ds(base, _BPW), pl.ds(t * E, E)],
            wsems.at[b])

    for t in range(F):
        b = t % _NB
        if t >= _NB and wd[t - _NB] is not None:
            wd[t - _NB].wait()
        if ABLATE_GATHER:
            gd[t] = pltpu.async_copy(tables[t].at[pl.ds(0, 1)],
                                     bufs_v.at[b, pl.ds(0, 1)], gsems.at[b])
        else:
            gd[t] = pltpu.async_copy(tables[t].at[idx_v.at[t]], bufs_v.at[b],
                                     gsems.at[b])
        if t - _LA >= 0:
            start_writeback(t - _LA)
    for t in range(F - _LA, F):
        start_writeback(t)
    for t in range(F - _NB, F):
        if wd[t] is not None:
            wd[t].wait()
    if ABLATE_WRITE:
        for t in range(F):
            gd[t].wait()


_emb = pl.kernel(
    _emb_body,
    mesh=plsc.VectorSubcoreMesh(core_axis_name="c", subcore_axis_name="s"),
    out_type=jax.ShapeDtypeStruct((B, F * E), jnp.float32),
    scratch_types=[
        pltpu.VMEM((F, _BPW), jnp.int32),
        pltpu.VMEM((_NB, _BPW, E), jnp.float32),
        pltpu.SemaphoreType.DMA,
        pltpu.SemaphoreType.DMA((_NB,)),
        pltpu.SemaphoreType.DMA((_NB,)),
    ],
    compiler_params=pltpu.CompilerParams(use_tc_tiling_on_sc=False),
)


@jax.jit
def kernel(id, age, pvalue, shop, occu, city, gender, cms,
           W_id, W_age, W_pvalue, W_shop, W_occu, W_city, W_gender, W_cms):
    out = _emb(id, age, pvalue, shop, occu, city, gender, cms,
               W_id, W_age, W_pvalue, W_shop, W_occu, W_city, W_gender, W_cms)
    return out.reshape(B, F, E)


# speedup vs baseline: 4.3377x; 4.3377x over previous
import jax
import jax.numpy as jnp
from jax import lax
from jax.experimental import pallas as pl
from jax.experimental.pallas import tpu as pltpu
from jax.experimental.pallas import tpu_sc as plsc

B = 16384
E = 64
F = 8

_NC = 2
_NS = 16
_NW = _NC * _NS
_BPW = B // _NW
_NCHUNK = 8
_CH = _BPW // _NCHUNK

_SMALL_OFFS = (0, 7, 11, 14, 16, 21, 23)
_SMALL_ROWS = 36


def _emb_body(id_h, age_h, pvalue_h, shop_h, occu_h, city_h, gender_h, cms_h,
              w_id_h, w_small_h, out_h,
              idx_v, ws_v, idbuf_v, asm_v, spidx_sh, sidx_m,
              isem, ssem, gsems, wsems):
    cid = lax.axis_index("c")
    sid = lax.axis_index("s")
    wid = sid * _NC + cid
    base = wid * _BPW
    idx_hbm = (age_h, pvalue_h, shop_h, occu_h, city_h, gender_h, cms_h)

    pltpu.sync_copy(id_h.at[pl.ds(base, _BPW)], idx_v.at[F - 1])

    gd = [pltpu.async_copy(w_id_h.at[idx_v.at[F - 1, pl.ds(k * _CH, _CH)]],
                           idbuf_v.at[k], gsems.at[k])
          for k in range(_NCHUNK)]

    icopies = [pltpu.async_copy(idx_hbm[f].at[pl.ds(base, _BPW)],
                                idx_v.at[f], isem) for f in range(F - 1)]
    scopy = pltpu.async_copy(w_small_h, ws_v, ssem)
    for c in icopies:
        c.wait()
    scopy.wait()
    bcopies = [pltpu.async_copy(idx_v.at[f, pl.ds(k * _CH, _CH)],
                                spidx_sh.at[sid, k, f], isem)
               for k in range(_NCHUNK) for f in range(F - 1)]
    for c in bcopies:
        c.wait()

    wd = [None] * _NCHUNK
    for k in range(_NCHUNK):
        p = k % 2
        pltpu.sync_copy(spidx_sh.at[sid, k], sidx_m)
        if k >= 2:
            wd[k - 2].wait()
        gd[k].wait()
        asm_p = asm_v.at[p]

        def row_body(i, _, asm_p=asm_p, k=k):
            for c in range(E // 16):
                asm_p[i, pl.ds(c * 16, 16)] = idbuf_v[k, i, pl.ds(c * 16, 16)]
            for f in range(F - 1):
                s = lax.min(lax.max(sidx_m[f, i], 0),
                            _SMALL_ROWS - 1 - _SMALL_OFFS[f]) + _SMALL_OFFS[f]
                for c in range(E // 16):
                    asm_p[i, pl.ds((f + 1) * E + c * 16, 16)] = \
                        ws_v[s, pl.ds(c * 16, 16)]
            return _

        lax.fori_loop(0, _CH, row_body, None)
        wd[k] = pltpu.async_copy(asm_p,
                                 out_h.at[pl.ds(base + k * _CH, _CH)],
                                 wsems.at[p])
    wd[_NCHUNK - 2].wait()
    wd[_NCHUNK - 1].wait()


_emb = pl.kernel(
    _emb_body,
    mesh=plsc.VectorSubcoreMesh(core_axis_name="c", subcore_axis_name="s"),
    out_type=jax.ShapeDtypeStruct((B, F * E), jnp.float32),
    scratch_types=[
        pltpu.VMEM((F, _BPW), jnp.int32),
        pltpu.VMEM((_SMALL_ROWS, E), jnp.float32),
        pltpu.VMEM((_NCHUNK, _CH, E), jnp.float32),
        pltpu.VMEM((2, _CH, F * E), jnp.float32),
        pltpu.VMEM_SHARED((_NS, _NCHUNK, F - 1, _CH), jnp.int32),
        pltpu.SMEM((F - 1, _CH), jnp.int32),
        pltpu.SemaphoreType.DMA,
        pltpu.SemaphoreType.DMA,
        pltpu.SemaphoreType.DMA((_NCHUNK,)),
        pltpu.SemaphoreType.DMA((2,)),
    ],
    compiler_params=pltpu.CompilerParams(use_tc_tiling_on_sc=False),
)


@jax.jit
def kernel(id, age, pvalue, shop, occu, city, gender, cms,
           W_id, W_age, W_pvalue, W_shop, W_occu, W_city, W_gender, W_cms):
    w_small = jnp.concatenate(
        [W_age, W_pvalue, W_shop, W_occu, W_city, W_gender, W_cms], axis=0)
    out = _emb(id, age, pvalue, shop, occu, city, gender, cms, W_id, w_small)
    return out.reshape(B, F, E)
